# Initial kernel scaffold; baseline (speedup 1.0000x reference)
import jax
import jax.numpy as jnp
from jax.experimental import pallas as pl


def kernel(prediction, target, no_bg):
    raise NotImplementedError("write your pallas kernel here")


# speedup vs baseline: 5.0414x; 5.0414x over previous
import jax
import jax.numpy as jnp
from jax import lax
from jax.experimental import pallas as pl
from jax.experimental.pallas import tpu as pltpu

_ROWS = 1152
_LANES = 128
_N = _ROWS * _LANES
_NI = 27


def _loss_body(nobg_ref, pred_ref, tgt_ref, out_ref,
               stats_ref, acc_ref, tot_ref, sid_ref):
    b = pl.program_id(0)
    ph = pl.program_id(1)
    j = pl.program_id(2)
    f32 = jnp.float32

    @pl.when(jnp.logical_and(ph == 0, j == 0))
    def _init():
        sid_ref[...] = tgt_ref[0, 0] * 9 + tgt_ref[0, 1] * 3 + tgt_ref[0, 2]

    @pl.when(jnp.logical_and(b == 0, jnp.logical_and(ph == 0, j == 0)))
    def _init_tot():
        tot_ref[0] = f32(0.0)

    @pl.when(ph == 0)
    def _stats():
        mf = (sid_ref[...] == j).astype(f32)
        stats_ref[j, 0] = jnp.sum(mf)
        stats_ref[j, 1] = jnp.sum(mf * pred_ref[0, 0])
        stats_ref[j, 2] = jnp.sum(mf * pred_ref[0, 1])
        stats_ref[j, 3] = jnp.sum(mf * pred_ref[0, 2])

    @pl.when(ph == 1)
    def _dense():
        x = pred_ref[0, 0]
        y = pred_ref[0, 1]
        z = pred_ref[0, 2]
        sid = sid_ref[...]
        cnt = stats_ref[j, 0]
        safe = jnp.maximum(cnt, 1.0)
        bg = j == 0
        mex = jnp.where(bg, 0.0, stats_ref[j, 1] / safe)
        mey = jnp.where(bg, 0.0, stats_ref[j, 2] / safe)
        mez = jnp.where(bg, 0.0, stats_ref[j, 3] / safe)
        m = sid == j
        dx = x - mex
        dy = y - mey
        dz = z - mez
        dx2 = dx * dx
        dy2 = dy * dy
        dz2 = dz * dz
        dist = dx2 + dy2 + dz2
        fr = 300.0 / (1.0 + dist)
        adx = jnp.abs(dx)
        ady = jnp.abs(dy)
        adz = jnp.abs(dz)
        hub = (jnp.where(adx < 1.0, 0.5 * dx2, adx - 0.5)
               + jnp.where(ady < 1.0, 0.5 * dy2, ady - 0.5)
               + jnp.where(adz < 1.0, 0.5 * dz2, adz - 0.5))
        zero = jnp.zeros_like(fr)
        Hj = jnp.sum(jnp.where(m, hub, zero))
        Sj = jnp.sum(fr)
        OWNj = jnp.sum(jnp.where(m, fr, zero))
        lanes = lax.broadcasted_iota(jnp.int32, (1, _LANES), 1)
        lm = lanes == j
        acc_ref[0:1] = jnp.where(lm, cnt, acc_ref[0:1])
        acc_ref[1:2] = jnp.where(lm, Hj, acc_ref[1:2])
        acc_ref[2:3] = jnp.where(lm, Sj, acc_ref[2:3])
        acc_ref[3:4] = jnp.where(lm, OWNj, acc_ref[3:4])
        acc_ref[4:5] = jnp.where(lm, mex, acc_ref[4:5])
        acc_ref[5:6] = jnp.where(lm, mey, acc_ref[5:6])
        acc_ref[6:7] = jnp.where(lm, mez, acc_ref[6:7])

        @pl.when(j == _NI - 1)
        def _assemble():
            lanes1 = lax.broadcasted_iota(jnp.int32, (1, _LANES), 1)
            inrange = lanes1 < _NI
            nobg_ok = nobg_ref[b] == 0
            cntv = acc_ref[0:1]
            Hv = acc_ref[1:2]
            Sv = acc_ref[2:3]
            OWNv = acc_ref[3:4]
            safev = jnp.maximum(cntv, 1.0)
            presentv = jnp.logical_and(cntv > 0.0, inrange)
            hmask = jnp.logical_and(presentv,
                                    jnp.logical_or(lanes1 > 0, nobg_ok))
            hterm = Hv / (safev * 3.0)
            ncv = f32(_N) - cntv
            sepv = ((Sv - OWNv) / jnp.maximum(ncv, 1.0)) * (10.0 / jnp.sqrt(safev))
            sepmask = jnp.logical_and(
                jnp.logical_and(presentv, ncv > 0.0), lanes1 > 0)
            zl = jnp.zeros_like(hterm)
            vv = jnp.where(hmask, jnp.ones_like(hterm), zl)
            loss = jnp.sum(jnp.where(hmask, hterm, zl)
                           + jnp.where(sepmask, sepv, zl))
            ct = jnp.sum(vv)

            onesv = jnp.ones((1, _LANES), f32)

            def _outer(v):
                return lax.dot_general(v, onesv, (((0,), (0,)), ((), ())),
                                       precision=lax.Precision.HIGHEST,
                                       preferred_element_type=f32)

            mxv = acc_ref[4:5]
            myv = acc_ref[5:6]
            mzv = acc_ref[6:7]
            ddx = _outer(mxv) - jnp.broadcast_to(mxv, (_LANES, _LANES))
            ddy = _outer(myv) - jnp.broadcast_to(myv, (_LANES, _LANES))
            ddz = _outer(mzv) - jnp.broadcast_to(mzv, (_LANES, _LANES))
            sq = ddx * ddx + ddy * ddy + ddz * ddz
            vcol = _outer(vv)
            vrow = jnp.broadcast_to(vv, (_LANES, _LANES))
            pv = vcol * vrow
            ri = lax.broadcasted_iota(jnp.int32, (_LANES, _LANES), 0)
            ci = lax.broadcasted_iota(jnp.int32, (_LANES, _LANES), 1)
            upper = jnp.logical_and(ri < ci, ci < _NI)
            zz = jnp.zeros_like(sq)
            pair_sum = jnp.sum(jnp.where(upper, (300.0 / (sq + 1.0)) * pv, zz))
            npair = jnp.sum(jnp.where(upper, pv, zz))
            pair_term = pair_sum / jnp.maximum(npair, 1.0)

            lossb = loss + jnp.where(ct > 1.0, pair_term, 0.0)
            tot_ref[0] += lossb / jnp.maximum(ct, 1.0)

            @pl.when(b == 1)
            def _finish():
                out_ref[...] = jnp.full((8, _LANES), tot_ref[0] * 0.5, f32)


def _make_call(interpret=False):
    return pl.pallas_call(
        _loss_body,
        grid=(2, 2, _NI),
        out_shape=jax.ShapeDtypeStruct((8, _LANES), jnp.float32),
        in_specs=[
            pl.BlockSpec(memory_space=pltpu.SMEM),
            pl.BlockSpec((1, 3, _ROWS, _LANES), lambda b, p, j: (b, 0, 0, 0)),
            pl.BlockSpec((1, 3, _ROWS, _LANES), lambda b, p, j: (b, 0, 0, 0)),
        ],
        out_specs=pl.BlockSpec((8, _LANES), lambda b, p, j: (0, 0)),
        scratch_shapes=[
            pltpu.SMEM((32, 4), jnp.float32),
            pltpu.VMEM((8, _LANES), jnp.float32),
            pltpu.SMEM((1,), jnp.float32),
            pltpu.VMEM((_ROWS, _LANES), jnp.int32),
        ],
        interpret=interpret,
    )


def kernel(prediction, target, no_bg):
    pred = prediction.astype(jnp.float32).reshape(2, 3, _ROWS, _LANES)
    tgt = target.astype(jnp.int32).reshape(2, 3, _ROWS, _LANES)
    nobg = no_bg.astype(jnp.int32)
    out = _make_call()(nobg, pred, tgt)
    return out[0, 0]


# speedup vs baseline: 8.1845x; 1.6235x over previous
import jax
import jax.numpy as jnp
from jax import lax
from jax.experimental import pallas as pl
from jax.experimental.pallas import tpu as pltpu

_ROWS = 1152
_LANES = 128
_N = _ROWS * _LANES
_NI = 27
_CHUNK = 32
_NCH = _ROWS // _CHUNK


def _loss_body(nobg_ref, pred_ref, tgt_ref, out_ref,
               stats_ref, acc_ref, tot_ref, sid_ref, hub_ref):
    b = pl.program_id(0)
    i = pl.program_id(1)
    f32 = jnp.float32

    @pl.when(i == 0)
    def _init():
        sid_ref[...] = tgt_ref[0, 0] * 9 + tgt_ref[0, 1] * 3 + tgt_ref[0, 2]

    @pl.when(jnp.logical_and(b == 0, i == 0))
    def _init_tot():
        tot_ref[0] = f32(0.0)

    @pl.when(i < _NI)
    def _stats():
        zc = jnp.zeros((_CHUNK, _LANES), f32)
        ca, xa, ya, za = zc, zc, zc, zc
        one = jnp.ones((_CHUNK, _LANES), f32)
        for c in range(_NCH):
            sl = pl.ds(c * _CHUNK, _CHUNK)
            m = sid_ref[sl] == i
            ca = ca + jnp.where(m, one, zc)
            xa = xa + jnp.where(m, pred_ref[0, 0, sl], zc)
            ya = ya + jnp.where(m, pred_ref[0, 1, sl], zc)
            za = za + jnp.where(m, pred_ref[0, 2, sl], zc)
        stats_ref[i, 0] = jnp.sum(ca)
        stats_ref[i, 1] = jnp.sum(xa)
        stats_ref[i, 2] = jnp.sum(ya)
        stats_ref[i, 3] = jnp.sum(za)

    @pl.when(i == _NI)
    def _gather_huber():
        mus = [(f32(0.0), f32(0.0), f32(0.0))]
        stats_ref[0, 4] = f32(0.0)
        stats_ref[0, 5] = f32(0.0)
        stats_ref[0, 6] = f32(0.0)
        for j in range(1, _NI):
            safe = jnp.maximum(stats_ref[j, 0], 1.0)
            mj = (stats_ref[j, 1] / safe,
                  stats_ref[j, 2] / safe,
                  stats_ref[j, 3] / safe)
            stats_ref[j, 4] = mj[0]
            stats_ref[j, 5] = mj[1]
            stats_ref[j, 6] = mj[2]
            mus.append(mj)
        for c in range(_NCH):
            sl = pl.ds(c * _CHUNK, _CHUNK)
            sid = sid_ref[sl]
            zc = jnp.zeros((_CHUNK, _LANES), f32)
            mx, my, mz = zc, zc, zc
            for j in range(1, _NI):
                m = sid == j
                mx = jnp.where(m, mus[j][0], mx)
                my = jnp.where(m, mus[j][1], my)
                mz = jnp.where(m, mus[j][2], mz)
            dx = pred_ref[0, 0, sl] - mx
            dy = pred_ref[0, 1, sl] - my
            dz = pred_ref[0, 2, sl] - mz
            adx = jnp.abs(dx)
            ady = jnp.abs(dy)
            adz = jnp.abs(dz)
            nx = jnp.minimum(adx, 1.0)
            ny = jnp.minimum(ady, 1.0)
            nz = jnp.minimum(adz, 1.0)
            hub = (nx * (2.0 * adx - nx) + ny * (2.0 * ady - ny)
                   + nz * (2.0 * adz - nz))
            hub_ref[sl] = 0.5 * hub

    @pl.when(i > _NI)
    def _dense():
        j = i - (_NI + 1)
        cnt = stats_ref[j, 0]
        mex = stats_ref[j, 4]
        mey = stats_ref[j, 5]
        mez = stats_ref[j, 6]
        zc = jnp.zeros((_CHUNK, _LANES), f32)
        sa, ha, oa = zc, zc, zc
        for c in range(_NCH):
            sl = pl.ds(c * _CHUNK, _CHUNK)
            m = sid_ref[sl] == j
            dx = pred_ref[0, 0, sl] - mex
            dy = pred_ref[0, 1, sl] - mey
            dz = pred_ref[0, 2, sl] - mez
            dist = dx * dx + dy * dy + dz * dz
            fr = 300.0 / (1.0 + dist)
            sa = sa + fr
            ha = ha + jnp.where(m, hub_ref[sl], zc)
            oa = oa + jnp.where(m, fr, zc)
        Sj = jnp.sum(sa)
        Hj = jnp.sum(ha)
        OWNj = jnp.sum(oa)
        lanes = lax.broadcasted_iota(jnp.int32, (1, _LANES), 1)
        lm = lanes == j
        acc_ref[0:1] = jnp.where(lm, cnt, acc_ref[0:1])
        acc_ref[1:2] = jnp.where(lm, Hj, acc_ref[1:2])
        acc_ref[2:3] = jnp.where(lm, Sj, acc_ref[2:3])
        acc_ref[3:4] = jnp.where(lm, OWNj, acc_ref[3:4])
        acc_ref[4:5] = jnp.where(lm, mex, acc_ref[4:5])
        acc_ref[5:6] = jnp.where(lm, mey, acc_ref[5:6])
        acc_ref[6:7] = jnp.where(lm, mez, acc_ref[6:7])

        @pl.when(j == _NI - 1)
        def _assemble():
            lanes1 = lax.broadcasted_iota(jnp.int32, (1, _LANES), 1)
            inrange = lanes1 < _NI
            nobg_ok = nobg_ref[b] == 0
            cntv = acc_ref[0:1]
            Hv = acc_ref[1:2]
            Sv = acc_ref[2:3]
            OWNv = acc_ref[3:4]
            safev = jnp.maximum(cntv, 1.0)
            presentv = jnp.logical_and(cntv > 0.0, inrange)
            hmask = jnp.logical_and(presentv,
                                    jnp.logical_or(lanes1 > 0, nobg_ok))
            hterm = Hv / (safev * 3.0)
            ncv = f32(_N) - cntv
            sepv = ((Sv - OWNv) / jnp.maximum(ncv, 1.0)) * (10.0 / jnp.sqrt(safev))
            sepmask = jnp.logical_and(
                jnp.logical_and(presentv, ncv > 0.0), lanes1 > 0)
            zl = jnp.zeros_like(hterm)
            vv = jnp.where(hmask, jnp.ones_like(hterm), zl)
            loss = jnp.sum(jnp.where(hmask, hterm, zl)
                           + jnp.where(sepmask, sepv, zl))
            ct = jnp.sum(vv)

            onesv = jnp.ones((1, _LANES), f32)

            def _outer(v):
                return lax.dot_general(v, onesv, (((0,), (0,)), ((), ())),
                                       precision=lax.Precision.HIGHEST,
                                       preferred_element_type=f32)

            mxv = acc_ref[4:5]
            myv = acc_ref[5:6]
            mzv = acc_ref[6:7]
            ddx = _outer(mxv) - jnp.broadcast_to(mxv, (_LANES, _LANES))
            ddy = _outer(myv) - jnp.broadcast_to(myv, (_LANES, _LANES))
            ddz = _outer(mzv) - jnp.broadcast_to(mzv, (_LANES, _LANES))
            sq = ddx * ddx + ddy * ddy + ddz * ddz
            vcol = _outer(vv)
            vrow = jnp.broadcast_to(vv, (_LANES, _LANES))
            pv = vcol * vrow
            ri = lax.broadcasted_iota(jnp.int32, (_LANES, _LANES), 0)
            ci = lax.broadcasted_iota(jnp.int32, (_LANES, _LANES), 1)
            upper = jnp.logical_and(ri < ci, ci < _NI)
            zz = jnp.zeros_like(sq)
            pair_sum = jnp.sum(jnp.where(upper, (300.0 / (sq + 1.0)) * pv, zz))
            npair = jnp.sum(jnp.where(upper, pv, zz))
            pair_term = pair_sum / jnp.maximum(npair, 1.0)

            lossb = loss + jnp.where(ct > 1.0, pair_term, 0.0)
            tot_ref[0] += lossb / jnp.maximum(ct, 1.0)

            @pl.when(b == 1)
            def _finish():
                out_ref[...] = jnp.full((8, _LANES), tot_ref[0] * 0.5, f32)


def _make_call(interpret=False):
    return pl.pallas_call(
        _loss_body,
        grid=(2, 2 * _NI + 1),
        out_shape=jax.ShapeDtypeStruct((8, _LANES), jnp.float32),
        in_specs=[
            pl.BlockSpec(memory_space=pltpu.SMEM),
            pl.BlockSpec((1, 3, _ROWS, _LANES), lambda b, i: (b, 0, 0, 0)),
            pl.BlockSpec((1, 3, _ROWS, _LANES), lambda b, i: (b, 0, 0, 0)),
        ],
        out_specs=pl.BlockSpec((8, _LANES), lambda b, i: (0, 0)),
        scratch_shapes=[
            pltpu.SMEM((32, 8), jnp.float32),
            pltpu.VMEM((8, _LANES), jnp.float32),
            pltpu.SMEM((1,), jnp.float32),
            pltpu.VMEM((_ROWS, _LANES), jnp.int32),
            pltpu.VMEM((_ROWS, _LANES), jnp.float32),
        ],
        interpret=interpret,
    )


def kernel(prediction, target, no_bg):
    pred = prediction.astype(jnp.float32).reshape(2, 3, _ROWS, _LANES)
    tgt = target.astype(jnp.int32).reshape(2, 3, _ROWS, _LANES)
    nobg = no_bg.astype(jnp.int32)
    out = _make_call()(nobg, pred, tgt)
    return out[0, 0]
